# baseline (speedup 1.0000x reference)
import functools

import jax
import jax.numpy as jnp
from jax import lax
from jax.experimental import pallas as pl
from jax.experimental.pallas import tpu as pltpu
from jax.experimental.pallas import tpu_sc as plsc

VOCAB = 100000
EMBED_DIM = 128
BATCH = 4096
SEQ = 200

NC, NS, L = 2, 16, 16
NW = NC * NS
B_TOTAL = BATCH * SEQ
B_PER_W = B_TOTAL // NW
GRP = 128
NGRP = B_PER_W // GRP
NBUF = 5
NIT = NGRP // NBUF
SPMEM = (False, False, False, False, True)
NSP = sum(SPMEM)


@functools.partial(
    pl.kernel,
    out_type=jax.ShapeDtypeStruct((B_TOTAL, EMBED_DIM), jnp.float32),
    mesh=plsc.VectorSubcoreMesh(core_axis_name="c", subcore_axis_name="s"),
    scratch_types=[
        pltpu.VMEM((NGRP, GRP), jnp.int32),
        pltpu.VMEM((NBUF, GRP, EMBED_DIM), jnp.float32),
        pltpu.VMEM_SHARED((NS, max(NSP, 1), GRP, EMBED_DIM), jnp.float32),
    ] + [pltpu.SemaphoreType.DMA] * (3 * NBUF),
)
def _gather_kernel(x_hbm, w_hbm, out_hbm, idx_v, rows_v, sp, *sems):
    sid = lax.axis_index("s")
    wid = sid * NC + lax.axis_index("c")
    base = wid * B_PER_W
    sems_g = sems[0:NBUF]
    sems_w = sems[NBUF:2 * NBUF]
    sems_c = sems[2 * NBUF:3 * NBUF]
    spslot = {}
    for b in range(NBUF):
        if SPMEM[b]:
            spslot[b] = len(spslot)
    pltpu.sync_copy(x_hbm.at[wid], idx_v)

    def fire_g(s, b):
        pltpu.async_copy(w_hbm.at[idx_v.at[s]], rows_v.at[b], sems_g[b])

    def wait_g(b):
        pltpu.make_async_copy(w_hbm.at[idx_v.at[0]], rows_v.at[b],
                              sems_g[b]).wait()

    def out_slice(s):
        return out_hbm.at[pl.ds(base + s * GRP, GRP)]

    def fire_wd(s, b):
        pltpu.async_copy(rows_v.at[b], out_slice(s), sems_w[b])

    def wait_wd(b):
        pltpu.make_async_copy(rows_v.at[b], out_slice(0), sems_w[b]).wait()

    def fire_c(b):
        pltpu.async_copy(rows_v.at[b], sp.at[sid, spslot[b]], sems_c[b])

    def wait_c(b):
        pltpu.make_async_copy(rows_v.at[b], sp.at[sid, spslot[b]],
                              sems_c[b]).wait()

    def fire_ws(s, b):
        pltpu.async_copy(sp.at[sid, spslot[b]], out_slice(s), sems_w[b])

    def wait_ws(b):
        pltpu.make_async_copy(sp.at[sid, spslot[b]], out_slice(0),
                              sems_w[b]).wait()

    for b in range(NBUF - 2):
        fire_g(b, b)

    def it(i, _):
        for j in range(NBUF):
            s = i * NBUF + j
            b2 = (j - 2) % NBUF

            @pl.when(s >= 2)
            def _():
                if SPMEM[b2]:
                    wait_c(b2)
                    fire_ws(s - 2, b2)
                else:
                    wait_wd(b2)

            @pl.when(s - 2 + NBUF < NGRP)
            def _():
                fire_g(s - 2 + NBUF, b2)

            wait_g(j)
            if SPMEM[j]:
                @pl.when(s >= NBUF)
                def _():
                    wait_ws(j)

                fire_c(j)
            else:
                fire_wd(s, j)
        return 0

    lax.fori_loop(0, NIT, it, 0)
    for k in range(2):
        b2 = (NGRP - 2 + k) % NBUF
        if SPMEM[b2]:
            wait_c(b2)
            fire_ws(NGRP - 2 + k, b2)
            wait_ws(b2)
        else:
            wait_wd(b2)


def kernel(x, W):
    x3 = x.reshape(NW, NGRP, GRP)
    out = _gather_kernel(x3, W)
    return out.reshape(BATCH, SEQ, EMBED_DIM)


# speedup vs baseline: 1.0321x; 1.0321x over previous
import functools

import jax
import jax.numpy as jnp
from jax import lax
from jax.experimental import pallas as pl
from jax.experimental.pallas import tpu as pltpu
from jax.experimental.pallas import tpu_sc as plsc

VOCAB = 100000
EMBED_DIM = 128
BATCH = 4096
SEQ = 200

NC, NS, L = 2, 16, 16
NW = NC * NS
B_TOTAL = BATCH * SEQ
B_PER_W = B_TOTAL // NW
GRP = 128
NGRP = B_PER_W // GRP
NBUF = 4
NIT = NGRP // NBUF
SPMEM = (False, False, True, True)
NSP = sum(SPMEM)


@functools.partial(
    pl.kernel,
    out_type=jax.ShapeDtypeStruct((B_TOTAL, EMBED_DIM), jnp.float32),
    mesh=plsc.VectorSubcoreMesh(core_axis_name="c", subcore_axis_name="s"),
    scratch_types=[
        pltpu.VMEM((NGRP, GRP), jnp.int32),
        pltpu.VMEM((NBUF, GRP, EMBED_DIM), jnp.float32),
        pltpu.VMEM_SHARED((NS, max(NSP, 1), GRP, EMBED_DIM), jnp.float32),
    ] + [pltpu.SemaphoreType.DMA] * (3 * NBUF),
)
def _gather_kernel(x_hbm, w_hbm, out_hbm, idx_v, rows_v, sp, *sems):
    sid = lax.axis_index("s")
    wid = sid * NC + lax.axis_index("c")
    base = wid * B_PER_W
    sems_g = sems[0:NBUF]
    sems_w = sems[NBUF:2 * NBUF]
    sems_c = sems[2 * NBUF:3 * NBUF]
    spslot = {}
    for b in range(NBUF):
        if SPMEM[b]:
            spslot[b] = len(spslot)
    pltpu.sync_copy(x_hbm.at[wid], idx_v)

    def fire_g(s, b):
        pltpu.async_copy(w_hbm.at[idx_v.at[s]], rows_v.at[b], sems_g[b])

    def wait_g(b):
        pltpu.make_async_copy(w_hbm.at[idx_v.at[0]], rows_v.at[b],
                              sems_g[b]).wait()

    def out_slice(s):
        return out_hbm.at[pl.ds(base + s * GRP, GRP)]

    def fire_wd(s, b):
        pltpu.async_copy(rows_v.at[b], out_slice(s), sems_w[b])

    def wait_wd(b):
        pltpu.make_async_copy(rows_v.at[b], out_slice(0), sems_w[b]).wait()

    def fire_c(b):
        pltpu.async_copy(rows_v.at[b], sp.at[sid, spslot[b]], sems_c[b])

    def wait_c(b):
        pltpu.make_async_copy(rows_v.at[b], sp.at[sid, spslot[b]],
                              sems_c[b]).wait()

    def fire_ws(s, b):
        pltpu.async_copy(sp.at[sid, spslot[b]], out_slice(s), sems_w[b])

    def wait_ws(b):
        pltpu.make_async_copy(sp.at[sid, spslot[b]], out_slice(0),
                              sems_w[b]).wait()

    for b in range(NBUF - 2):
        fire_g(b, b)

    def it(i, _):
        for j in range(NBUF):
            s = i * NBUF + j
            b2 = (j - 2) % NBUF

            @pl.when(s >= 2)
            def _():
                if SPMEM[b2]:
                    wait_c(b2)
                    fire_ws(s - 2, b2)
                else:
                    wait_wd(b2)

            @pl.when(s - 2 + NBUF < NGRP)
            def _():
                fire_g(s - 2 + NBUF, b2)

            wait_g(j)
            if SPMEM[j]:
                @pl.when(s >= NBUF)
                def _():
                    wait_ws(j)

                fire_c(j)
            else:
                fire_wd(s, j)
        return 0

    lax.fori_loop(0, NIT, it, 0)
    for k in range(2):
        b2 = (NGRP - 2 + k) % NBUF
        if SPMEM[b2]:
            wait_c(b2)
            fire_ws(NGRP - 2 + k, b2)
            wait_ws(b2)
        else:
            wait_wd(b2)


def kernel(x, W):
    x3 = x.reshape(NW, NGRP, GRP)
    out = _gather_kernel(x3, W)
    return out.reshape(BATCH, SEQ, EMBED_DIM)


# speedup vs baseline: 1.0351x; 1.0029x over previous
import functools

import jax
import jax.numpy as jnp
from jax import lax
from jax.experimental import pallas as pl
from jax.experimental.pallas import tpu as pltpu
from jax.experimental.pallas import tpu_sc as plsc

VOCAB = 100000
EMBED_DIM = 128
BATCH = 4096
SEQ = 200

NC, NS, L = 2, 16, 16
NW = NC * NS
B_TOTAL = BATCH * SEQ
B_PER_W = B_TOTAL // NW
GRP = 128
NGRP = B_PER_W // GRP
NBUF = 4
NIT = NGRP // NBUF
SPMEM = (False, True, False, True)
NSP = sum(SPMEM)


@functools.partial(
    pl.kernel,
    out_type=jax.ShapeDtypeStruct((B_TOTAL, EMBED_DIM), jnp.float32),
    mesh=plsc.VectorSubcoreMesh(core_axis_name="c", subcore_axis_name="s"),
    scratch_types=[
        pltpu.VMEM((NGRP, GRP), jnp.int32),
        pltpu.VMEM((NBUF, GRP, EMBED_DIM), jnp.float32),
        pltpu.VMEM_SHARED((NS, max(NSP, 1), GRP, EMBED_DIM), jnp.float32),
    ] + [pltpu.SemaphoreType.DMA] * (3 * NBUF),
)
def _gather_kernel(x_hbm, w_hbm, out_hbm, idx_v, rows_v, sp, *sems):
    sid = lax.axis_index("s")
    wid = sid * NC + lax.axis_index("c")
    base = wid * B_PER_W
    sems_g = sems[0:NBUF]
    sems_w = sems[NBUF:2 * NBUF]
    sems_c = sems[2 * NBUF:3 * NBUF]
    spslot = {}
    for b in range(NBUF):
        if SPMEM[b]:
            spslot[b] = len(spslot)
    pltpu.sync_copy(x_hbm.at[wid], idx_v)

    def fire_g(s, b):
        pltpu.async_copy(w_hbm.at[idx_v.at[s]], rows_v.at[b], sems_g[b])

    def wait_g(b):
        pltpu.make_async_copy(w_hbm.at[idx_v.at[0]], rows_v.at[b],
                              sems_g[b]).wait()

    def out_slice(s):
        return out_hbm.at[pl.ds(base + s * GRP, GRP)]

    def fire_wd(s, b):
        pltpu.async_copy(rows_v.at[b], out_slice(s), sems_w[b])

    def wait_wd(b):
        pltpu.make_async_copy(rows_v.at[b], out_slice(0), sems_w[b]).wait()

    def fire_c(b):
        pltpu.async_copy(rows_v.at[b], sp.at[sid, spslot[b]], sems_c[b])

    def wait_c(b):
        pltpu.make_async_copy(rows_v.at[b], sp.at[sid, spslot[b]],
                              sems_c[b]).wait()

    def fire_ws(s, b):
        pltpu.async_copy(sp.at[sid, spslot[b]], out_slice(s), sems_w[b])

    def wait_ws(b):
        pltpu.make_async_copy(sp.at[sid, spslot[b]], out_slice(0),
                              sems_w[b]).wait()

    for b in range(NBUF - 2):
        fire_g(b, b)

    def it(i, _):
        for j in range(NBUF):
            s = i * NBUF + j
            b2 = (j - 2) % NBUF

            @pl.when(s >= 2)
            def _():
                if SPMEM[b2]:
                    wait_c(b2)
                    fire_ws(s - 2, b2)
                else:
                    wait_wd(b2)

            @pl.when(s - 2 + NBUF < NGRP)
            def _():
                fire_g(s - 2 + NBUF, b2)

            wait_g(j)
            if SPMEM[j]:
                @pl.when(s >= NBUF)
                def _():
                    wait_ws(j)

                fire_c(j)
            else:
                fire_wd(s, j)
        return 0

    lax.fori_loop(0, NIT, it, 0)
    for k in range(2):
        b2 = (NGRP - 2 + k) % NBUF
        if SPMEM[b2]:
            wait_c(b2)
            fire_ws(NGRP - 2 + k, b2)
            wait_ws(b2)
        else:
            wait_wd(b2)


def kernel(x, W):
    x3 = x.reshape(NW, NGRP, GRP)
    out = _gather_kernel(x3, W)
    return out.reshape(BATCH, SEQ, EMBED_DIM)


# speedup vs baseline: 1.0368x; 1.0017x over previous
import functools

import jax
import jax.numpy as jnp
from jax import lax
from jax.experimental import pallas as pl
from jax.experimental.pallas import tpu as pltpu
from jax.experimental.pallas import tpu_sc as plsc

VOCAB = 100000
EMBED_DIM = 128
BATCH = 4096
SEQ = 200

NC, NS, L = 2, 16, 16
NW = NC * NS
B_TOTAL = BATCH * SEQ
B_PER_W = B_TOTAL // NW
GRP = 128
NGRP = B_PER_W // GRP
NBUF = 4
NIT = NGRP // NBUF
SPMEM = (False, True, False, True)
NSP = sum(SPMEM)


@functools.partial(
    pl.kernel,
    out_type=jax.ShapeDtypeStruct((B_TOTAL, EMBED_DIM), jnp.float32),
    mesh=plsc.VectorSubcoreMesh(core_axis_name="c", subcore_axis_name="s"),
    scratch_types=[
        pltpu.VMEM((NGRP, GRP), jnp.int32),
        pltpu.VMEM((NBUF, GRP, EMBED_DIM), jnp.float32),
        pltpu.VMEM_SHARED((NS, max(NSP, 1), GRP, EMBED_DIM), jnp.float32),
    ] + [pltpu.SemaphoreType.DMA] * (3 * NBUF),
)
def _gather_kernel(x_hbm, w_hbm, out_hbm, idx_v, rows_v, sp, *sems):
    sid = lax.axis_index("s")
    wid = sid * NC + lax.axis_index("c")
    base = wid * B_PER_W
    sems_g = sems[0:NBUF]
    sems_w = sems[NBUF:2 * NBUF]
    sems_c = sems[2 * NBUF:3 * NBUF]
    spslot = {}
    for b in range(NBUF):
        if SPMEM[b]:
            spslot[b] = len(spslot)
    pltpu.sync_copy(x_hbm.at[wid], idx_v)

    def fire_g(s, b):
        pltpu.async_copy(w_hbm.at[idx_v.at[s]], rows_v.at[b], sems_g[b])

    def wait_g(b):
        pltpu.make_async_copy(w_hbm.at[idx_v.at[0]], rows_v.at[b],
                              sems_g[b]).wait()

    def out_slice(s):
        return out_hbm.at[pl.ds(base + s * GRP, GRP)]

    def fire_wd(s, b):
        pltpu.async_copy(rows_v.at[b], out_slice(s), sems_w[b])

    def wait_wd(b):
        pltpu.make_async_copy(rows_v.at[b], out_slice(0), sems_w[b]).wait()

    def fire_c(b):
        pltpu.async_copy(rows_v.at[b], sp.at[sid, spslot[b]], sems_c[b])

    def wait_c(b):
        pltpu.make_async_copy(rows_v.at[b], sp.at[sid, spslot[b]],
                              sems_c[b]).wait()

    def fire_ws(s, b):
        pltpu.async_copy(sp.at[sid, spslot[b]], out_slice(s), sems_w[b])

    def wait_ws(b):
        pltpu.make_async_copy(sp.at[sid, spslot[b]], out_slice(0),
                              sems_w[b]).wait()

    for b in range(NBUF - 2):
        fire_g(b, b)

    def it(i, _):
        for j in range(NBUF):
            s = i * NBUF + j
            b2 = (j - 2) % NBUF

            @pl.when(s >= 2)
            def _():
                if SPMEM[b2]:
                    wait_c(b2)
                    fire_ws(s - 2, b2)
                else:
                    wait_wd(b2)

            @pl.when(s - 2 + NBUF < NGRP)
            def _():
                fire_g(s - 2 + NBUF, b2)

            wait_g(j)
            if SPMEM[j]:
                @pl.when(s >= NBUF)
                def _():
                    wait_ws(j)

                fire_c(j)
            else:
                fire_wd(s, j)
        return 0

    lax.fori_loop(0, NIT, it, 0)
    last_a_slots = {(NGRP - 2) % NBUF, (NGRP - 1) % NBUF}
    for b in range(NBUF):
        if SPMEM[b] and ((b + 2) % NBUF) in last_a_slots:
            wait_ws(b)
    for k in range(2):
        b2 = (NGRP - 2 + k) % NBUF
        if SPMEM[b2]:
            wait_c(b2)
            fire_ws(NGRP - 2 + k, b2)
            wait_ws(b2)
        else:
            wait_wd(b2)


def kernel(x, W):
    x3 = x.reshape(NW, NGRP, GRP)
    out = _gather_kernel(x3, W)
    return out.reshape(BATCH, SEQ, EMBED_DIM)
